# baseline (speedup 1.0000x reference)
import functools

import jax
import jax.numpy as jnp
from jax import lax
from jax.experimental import pallas as pl
from jax.experimental.pallas import tpu as pltpu
from jax.experimental.pallas import tpu_sc as plsc

_NS = 16
_CH = 128
_BN = 1000


def _tc_nbr_body(x_ref, wn_ref, oa_ref, ob_ref):
    hn = jnp.dot(x_ref[...], wn_ref[...], preferred_element_type=jnp.float32)
    oa_ref[...] = hn[:, :128]
    ob_ref[...] = hn[:, 128:]


def _tc_mid_body(x_ref, aa_ref, ab_ref, ws_ref, b_ref, g_ref, be_ref,
                 wn7_ref, h_ref, oa_ref, ob_ref):
    x = x_ref[...]
    agg = jnp.concatenate([aa_ref[...], ab_ref[...]], axis=1)
    c = jnp.dot(x, ws_ref[...], preferred_element_type=jnp.float32)
    c = c + agg + b_ref[...]
    mu = jnp.mean(c, axis=1, keepdims=True)
    var = jnp.mean((c - mu) ** 2, axis=1, keepdims=True)
    ln = (c - mu) * lax.rsqrt(var + 1e-5) * g_ref[...] + be_ref[...]
    h = x + jnp.maximum(ln, 0.0)
    h_ref[...] = h
    hn7 = jnp.dot(h, wn7_ref[...], preferred_element_type=jnp.float32)
    oa_ref[...] = hn7[:, :128]
    ob_ref[...] = hn7[:, 128:]


def _tc_out_body(h_ref, aa_ref, ab_ref, ws_ref, b_ref, g_ref, be_ref,
                 wp_ref, bp_ref, o_ref):
    h = h_ref[...]
    agg = jnp.concatenate([aa_ref[...], ab_ref[...]], axis=1)
    c = jnp.dot(h, ws_ref[...], preferred_element_type=jnp.float32)
    c = c + agg + b_ref[...]
    mu = jnp.mean(c, axis=1, keepdims=True)
    var = jnp.mean((c - mu) ** 2, axis=1, keepdims=True)
    ln = (c - mu) * lax.rsqrt(var + 1e-5) * g_ref[...] + be_ref[...]
    h2 = h + jnp.maximum(ln, 0.0)
    o_ref[...] = (jnp.dot(h2, wp_ref[...], preferred_element_type=jnp.float32)
                  + bp_ref[...])


def _row_spec(w):
    return pl.BlockSpec((_BN, w), lambda i: (i, 0))


def _full_spec(shape):
    return pl.BlockSpec(shape, lambda i: tuple(0 for _ in shape))


def _sc_agg_call(hn_a, hn_b, edges3, n_nodes):
    n_chunks = edges3.shape[1]
    n_acc = -(-n_nodes // (_NS * _CH)) * (_NS * _CH)
    npt = n_acc // _NS
    mesh = plsc.VectorSubcoreMesh(core_axis_name="c", subcore_axis_name="s")

    @functools.partial(
        pl.kernel,
        out_type=[jax.ShapeDtypeStruct((n_acc, 128), jnp.float32)] * 2,
        mesh=mesh,
        scratch_types=[
            pltpu.VMEM((_CH, 128), jnp.float32),
            pltpu.VMEM((_CH, 128), jnp.float32),
            pltpu.VMEM((3, _CH), jnp.int32),
            pltpu.VMEM((3, _CH), jnp.int32),
            pltpu.VMEM((3, _CH), jnp.int32),
            pltpu.VMEM((3, _CH), jnp.int32),
            pltpu.VMEM_SHARED((n_acc, 128), jnp.float32),
            pltpu.SemaphoreType.DMA,
            pltpu.SemaphoreType.DMA,
            pltpu.SemaphoreType.DMA,
            pltpu.SemaphoreType.DMA,
            pltpu.SemaphoreType.DMA,
            pltpu.SemaphoreType.DMA,
            pltpu.SemaphoreType.DMA,
            pltpu.SemaphoreType.DMA,
        ],
    )
    def sc_kernel(hn_a_hbm, hn_b_hbm, edges_hbm,
                  agg_a_hbm, agg_b_hbm,
                  rows0_v, rows1_v, ib0, ib1, ib2, ib3, acc_sh,
                  gsem0, gsem1, ssem0, ssem1, is0, is1, is2, is3):
        rows_v = rows0_v
        c = lax.axis_index("c")
        s = lax.axis_index("s")
        base = s * npt
        ibufs = (ib0, ib1, ib2, ib3)
        isems = (is0, is1, is2, is3)

        def _zrow(i, _):
            for k in range(8):
                rows_v[i, pl.ds(k * 16, 16)] = jnp.zeros((16,), jnp.float32)
            return 0
        lax.fori_loop(0, _CH, _zrow, 0)
        for t in range(npt // _CH):
            pltpu.sync_copy(rows_v,
                            acc_sh.at[pl.ds(base + t * _CH, _CH)])
        plsc.subcore_barrier()

        def _edges(hn_hbm):
            bufs = (rows0_v, rows1_v)
            gsems = (gsem0, gsem1)
            ssems = (ssem0, ssem1)

            def start_idx(j, q):
                pltpu.async_copy(edges_hbm.at[s].at[j], ibufs[q], isems[q])

            def wait_idx(j, q):
                pltpu.make_async_copy(
                    edges_hbm.at[s].at[j], ibufs[q], isems[q]).wait()

            def start_gather(j, q, b):
                pltpu.async_copy(hn_hbm.at[ibufs[q].at[0]], bufs[b], gsems[b])

            def wait_gather(j, q, b):
                pltpu.make_async_copy(
                    hn_hbm.at[ibufs[q].at[0]], bufs[b], gsems[b]).wait()

            def start_scatter(j, q, b):
                pltpu.async_copy(bufs[b], acc_sh.at[ibufs[q].at[1]], ssems[b],
                                 add=True)

            def wait_scatter(j, q, b):
                pltpu.make_async_copy(
                    bufs[b], acc_sh.at[ibufs[q].at[1]], ssems[b]).wait()

            def mul(q, b):
                buf = bufs[b]
                ib = ibufs[q]

                def group_body(g, _):
                    ew16i = ib[2, pl.ds(g * 16, 16)]
                    for e in range(16):
                        row = g * 16 + e
                        ewv = jnp.full(
                            (16,),
                            lax.bitcast_convert_type(ew16i[e], jnp.float32),
                            jnp.float32)
                        for k in range(8):
                            sl = buf[row, pl.ds(k * 16, 16)]
                            buf[row, pl.ds(k * 16, 16)] = sl * ewv
                    return 0
                lax.fori_loop(0, _CH // 16, group_body, 0)

            nt = n_chunks // 4
            for q in range(3):
                start_idx(q, q)
            wait_idx(0, 0)
            start_gather(0, 0, 0)

            def quad_body(t, _):
                j0 = 4 * t
                for q in range(4):
                    j = j0 + q
                    b = q & 1
                    qn = (q + 1) & 3
                    wait_gather(j, q, b)
                    if q == 0:
                        @pl.when(t > 0)
                        def _():
                            wait_scatter(j - 1, 3, 1 - b)
                    else:
                        wait_scatter(j - 1, q - 1, 1 - b)
                    if q == 3:
                        @pl.when(t < nt - 1)
                        def _():
                            wait_idx(j + 1, qn)
                            start_gather(j + 1, qn, 1 - b)
                    else:
                        wait_idx(j + 1, qn)
                        start_gather(j + 1, qn, 1 - b)
                    mul(q, b)
                    start_scatter(j, q, b)
                    if q == 0:
                        start_idx(j + 3, 3)
                    else:
                        @pl.when(t < nt - 1)
                        def _():
                            start_idx(j + 3, (q + 3) & 3)
                return 0
            lax.fori_loop(0, nt, quad_body, 0)
            wait_scatter(n_chunks - 1, 3, 1)

        @pl.when(c == 0)
        def _():
            _edges(hn_a_hbm)

        @pl.when(c == 1)
        def _():
            _edges(hn_b_hbm)

        plsc.subcore_barrier()

        def _copy_out(agg_hbm):
            for t in range(npt // _CH):
                sl = pl.ds(base + t * _CH, _CH)
                pltpu.sync_copy(acc_sh.at[sl], rows_v)
                pltpu.sync_copy(rows_v, agg_hbm.at[sl])

        @pl.when(c == 0)
        def _():
            _copy_out(agg_a_hbm)

        @pl.when(c == 1)
        def _():
            _copy_out(agg_b_hbm)

    return sc_kernel(hn_a, hn_b, edges3)


def kernel(x, edge_index, edge_weight, W6_self, W6_nbr, b6, g6, beta6,
           W7_self, W7_nbr, b7, g7, beta7, Wp, bp):
    n, d = x.shape
    e = edge_weight.shape[0]
    out_d = Wp.shape[1]
    grid = (n // _BN,)

    n_chunks = (e + _NS * _CH - 1) // (_NS * _CH)
    n_chunks += (-n_chunks) % 4
    e_pad = _NS * n_chunks * _CH
    src = edge_index[0]
    dst = edge_index[1]
    ew = edge_weight
    if e_pad != e:
        p = e_pad - e
        src = jnp.concatenate([src, jnp.zeros((p,), jnp.int32)])
        dst = jnp.concatenate([dst, jnp.full((p,), n, jnp.int32)])
        ew = jnp.concatenate([ew, jnp.zeros((p,), jnp.float32)])
    edges3 = jnp.stack(
        [src.reshape(_NS, n_chunks, _CH),
         dst.reshape(_NS, n_chunks, _CH),
         lax.bitcast_convert_type(ew, jnp.int32).reshape(_NS, n_chunks, _CH)],
        axis=2)

    b6r, g6r, be6r = b6.reshape(1, d), g6.reshape(1, d), beta6.reshape(1, d)
    b7r, g7r, be7r = b7.reshape(1, d), g7.reshape(1, d), beta7.reshape(1, d)
    bpr = bp.reshape(1, out_d)

    tc_nbr = pl.pallas_call(
        _tc_nbr_body,
        grid=grid,
        in_specs=[_row_spec(d), _full_spec((d, d))],
        out_specs=[_row_spec(128), _row_spec(128)],
        out_shape=[jax.ShapeDtypeStruct((n, 128), jnp.float32)] * 2,
    )

    tc_mid = pl.pallas_call(
        _tc_mid_body,
        grid=grid,
        in_specs=[_row_spec(d), _row_spec(128), _row_spec(128),
                  _full_spec((d, d)), _full_spec((1, d)), _full_spec((1, d)),
                  _full_spec((1, d)), _full_spec((d, d))],
        out_specs=[_row_spec(d), _row_spec(128), _row_spec(128)],
        out_shape=[jax.ShapeDtypeStruct((n, d), jnp.float32),
                   jax.ShapeDtypeStruct((n, 128), jnp.float32),
                   jax.ShapeDtypeStruct((n, 128), jnp.float32)],
    )

    tc_out = pl.pallas_call(
        _tc_out_body,
        grid=grid,
        in_specs=[_row_spec(d), _row_spec(128), _row_spec(128),
                  _full_spec((d, d)), _full_spec((1, d)), _full_spec((1, d)),
                  _full_spec((1, d)), _full_spec((d, out_d)),
                  _full_spec((1, out_d))],
        out_specs=pl.BlockSpec((_BN, out_d), lambda i: (i, 0)),
        out_shape=jax.ShapeDtypeStruct((n, out_d), jnp.float32),
    )

    hn6a, hn6b = tc_nbr(x, W6_nbr)
    agg6a, agg6b = _sc_agg_call(hn6a, hn6b, edges3, n)
    h, hn7a, hn7b = tc_mid(x, agg6a, agg6b, W6_self, b6r, g6r, be6r, W7_nbr)
    agg7a, agg7b = _sc_agg_call(hn7a, hn7b, edges3, n)
    return tc_out(h, agg7a, agg7b, W7_self, b7r, g7r, be7r, Wp, bpr)
